# Initial kernel scaffold; baseline (speedup 1.0000x reference)
import jax
import jax.numpy as jnp
from jax.experimental import pallas as pl


def kernel(intent_ids, table):
    raise NotImplementedError("write your pallas kernel here")


# speedup vs baseline: 4.7476x; 4.7476x over previous
import functools

import jax
import jax.numpy as jnp
from jax import lax
from jax.experimental import pallas as pl
from jax.experimental.pallas import tpu as pltpu
from jax.experimental.pallas import tpu_sc as plsc

BATCH = 16384
SEQ_LEN = 200
EMBED_DIM = 64
TOTAL = BATCH * SEQ_LEN

_info = plsc.get_sparse_core_info()
_NC = _info.num_cores
_NS = _info.num_subcores
_NW = _NC * _NS
_BPW = TOTAL // _NW

_CHUNK = 512
_NCHUNK = _BPW // _CHUNK

_mesh = plsc.VectorSubcoreMesh(core_axis_name="c", subcore_axis_name="s")


@functools.partial(
    pl.kernel,
    mesh=_mesh,
    out_type=jax.ShapeDtypeStruct((TOTAL, EMBED_DIM), jnp.float32),
    scratch_types=[
        pltpu.VMEM((_CHUNK,), jnp.int32),
        pltpu.VMEM((_CHUNK, EMBED_DIM), jnp.float32),
        pltpu.SemaphoreType.DMA,
    ],
    compiler_params=pltpu.CompilerParams(use_tc_tiling_on_sc=False),
)
def _gather_kernel(ids_hbm, table_hbm, out_hbm, idx_v, rows_v, sem):
    wid = lax.axis_index("s") * _NC + lax.axis_index("c")
    base = wid * _BPW

    def body(i, carry):
        off = base + i * _CHUNK
        pltpu.sync_copy(ids_hbm.at[pl.ds(off, _CHUNK)], idx_v)
        pltpu.async_copy(table_hbm.at[idx_v], rows_v, sem).wait()
        pltpu.sync_copy(rows_v, out_hbm.at[pl.ds(off, _CHUNK)])
        return carry

    lax.fori_loop(0, _NCHUNK, body, 0)


def kernel(intent_ids, table):
    ids = intent_ids.reshape(TOTAL).astype(jnp.int32)
    out = _gather_kernel(ids, table)
    return out.reshape(BATCH, SEQ_LEN, EMBED_DIM)


# speedup vs baseline: 5.1692x; 1.0888x over previous
import functools

import jax
import jax.numpy as jnp
from jax import lax
from jax.experimental import pallas as pl
from jax.experimental.pallas import tpu as pltpu
from jax.experimental.pallas import tpu_sc as plsc

BATCH = 16384
SEQ_LEN = 200
EMBED_DIM = 64
TOTAL = BATCH * SEQ_LEN

_info = plsc.get_sparse_core_info()
_NC = _info.num_cores
_NS = _info.num_subcores
_NW = _NC * _NS
_BPW = TOTAL // _NW

_CHUNK = 800
_NCHUNK = _BPW // _CHUNK
_NPAIR = _NCHUNK // 2

_mesh = plsc.VectorSubcoreMesh(core_axis_name="c", subcore_axis_name="s")


@functools.partial(
    pl.kernel,
    mesh=_mesh,
    out_type=jax.ShapeDtypeStruct((TOTAL, EMBED_DIM), jnp.float32),
    scratch_types=[
        pltpu.VMEM((_CHUNK,), jnp.int32),
        pltpu.VMEM((_CHUNK,), jnp.int32),
        pltpu.VMEM((_CHUNK, EMBED_DIM), jnp.float32),
        pltpu.VMEM((_CHUNK, EMBED_DIM), jnp.float32),
        pltpu.SemaphoreType.DMA,
        pltpu.SemaphoreType.DMA,
        pltpu.SemaphoreType.DMA,
        pltpu.SemaphoreType.DMA,
        pltpu.SemaphoreType.DMA,
        pltpu.SemaphoreType.DMA,
    ],
    compiler_params=pltpu.CompilerParams(use_tc_tiling_on_sc=False),
)
def _gather_kernel(ids_hbm, table_hbm, out_hbm, idx0, idx1, rows0, rows1,
                   s_idx0, s_idx1, s_gat0, s_gat1, s_out0, s_out1):
    wid = lax.axis_index("s") * _NC + lax.axis_index("c")
    base = wid * _BPW

    idx = (idx0, idx1)
    rows = (rows0, rows1)
    s_idx = (s_idx0, s_idx1)
    s_gat = (s_gat0, s_gat1)
    s_out = (s_out0, s_out1)

    def ids_at(i):
        return ids_hbm.at[pl.ds(base + i * _CHUNK, _CHUNK)]

    def out_at(i):
        return out_hbm.at[pl.ds(base + i * _CHUNK, _CHUNK)]

    def half(g, b, *, first=False, last=False, prefetch=True):
        ob = 1 - b
        i = 2 * g + b
        if not last:
            pltpu.make_async_copy(ids_at(i + 1), idx[ob], s_idx[ob]).wait()
        if not first:
            pltpu.make_async_copy(rows[ob], out_at(i - 1), s_out[ob]).wait()
        if not last:
            pltpu.async_copy(table_hbm.at[idx[ob]], rows[ob], s_gat[ob])
        pltpu.make_async_copy(table_hbm.at[idx[b]], rows[b], s_gat[b]).wait()
        pltpu.async_copy(rows[b], out_at(i), s_out[b])
        if not last and prefetch:
            pltpu.async_copy(ids_at(i + 2), idx[b], s_idx[b])

    def pair(g, carry):
        half(g, 0)
        half(g, 1)
        return carry

    pltpu.async_copy(ids_at(0), idx[0], s_idx[0])
    pltpu.async_copy(ids_at(1), idx[1], s_idx[1])
    pltpu.make_async_copy(ids_at(0), idx[0], s_idx[0]).wait()
    pltpu.async_copy(table_hbm.at[idx[0]], rows[0], s_gat[0])

    half(0, 0, first=True)
    half(0, 1)
    lax.fori_loop(1, _NPAIR - 1, pair, 0)
    half(_NPAIR - 1, 0, prefetch=False)
    half(_NPAIR - 1, 1, last=True)
    pltpu.make_async_copy(rows[1], out_at(_NCHUNK - 1), s_out[1]).wait()


def kernel(intent_ids, table):
    ids = intent_ids.reshape(TOTAL).astype(jnp.int32)
    out = _gather_kernel(ids, table)
    return out.reshape(BATCH, SEQ_LEN, EMBED_DIM)


# speedup vs baseline: 5.1783x; 1.0018x over previous
import functools

import jax
import jax.numpy as jnp
from jax import lax
from jax.experimental import pallas as pl
from jax.experimental.pallas import tpu as pltpu
from jax.experimental.pallas import tpu_sc as plsc

BATCH = 16384
SEQ_LEN = 200
EMBED_DIM = 64
TOTAL = BATCH * SEQ_LEN

_info = plsc.get_sparse_core_info()
_NC = _info.num_cores
_NS = _info.num_subcores
_NW = _NC * _NS
_ROWS_PW = BATCH // _NW

_CROWS = 4
_CHUNK = _CROWS * SEQ_LEN
_NCHUNK = _ROWS_PW // _CROWS
_NPAIR = _NCHUNK // 2

_mesh = plsc.VectorSubcoreMesh(core_axis_name="c", subcore_axis_name="s")


@functools.partial(
    pl.kernel,
    mesh=_mesh,
    out_type=jax.ShapeDtypeStruct((BATCH, SEQ_LEN, EMBED_DIM), jnp.float32),
    scratch_types=[
        pltpu.VMEM((_CHUNK,), jnp.int32),
        pltpu.VMEM((_CHUNK,), jnp.int32),
        pltpu.VMEM((_CROWS, SEQ_LEN, EMBED_DIM), jnp.float32),
        pltpu.VMEM((_CROWS, SEQ_LEN, EMBED_DIM), jnp.float32),
        pltpu.SemaphoreType.DMA,
        pltpu.SemaphoreType.DMA,
        pltpu.SemaphoreType.DMA,
        pltpu.SemaphoreType.DMA,
        pltpu.SemaphoreType.DMA,
        pltpu.SemaphoreType.DMA,
    ],
    compiler_params=pltpu.CompilerParams(use_tc_tiling_on_sc=False),
)
def _gather_kernel(ids_hbm, table_hbm, out_hbm, idx0, idx1, rows0, rows1,
                   s_idx0, s_idx1, s_gat0, s_gat1, s_out0, s_out1):
    wid = lax.axis_index("s") * _NC + lax.axis_index("c")
    base = wid * _ROWS_PW * SEQ_LEN
    rbase = wid * _ROWS_PW

    idx = (idx0, idx1)
    rows = (rows0, rows1)
    s_idx = (s_idx0, s_idx1)
    s_gat = (s_gat0, s_gat1)
    s_out = (s_out0, s_out1)

    def ids_at(i):
        return ids_hbm.at[pl.ds(base + i * _CHUNK, _CHUNK)]

    def out_at(i):
        return out_hbm.at[pl.ds(rbase + i * _CROWS, _CROWS)]

    def gat_start(b):
        for j in range(_CROWS):
            pltpu.async_copy(
                table_hbm.at[idx[b].at[pl.ds(j * SEQ_LEN, SEQ_LEN)]],
                rows[b].at[j], s_gat[b])

    def gat_wait(b):
        for j in range(_CROWS):
            pltpu.make_async_copy(
                table_hbm.at[idx[b].at[pl.ds(j * SEQ_LEN, SEQ_LEN)]],
                rows[b].at[j], s_gat[b]).wait()

    def half(g, b, *, first=False, last=False, prefetch=True):
        ob = 1 - b
        i = 2 * g + b
        if not last:
            pltpu.make_async_copy(ids_at(i + 1), idx[ob], s_idx[ob]).wait()
        if not first:
            pltpu.make_async_copy(rows[ob], out_at(i - 1), s_out[ob]).wait()
        if not last:
            gat_start(ob)
        gat_wait(b)
        pltpu.async_copy(rows[b], out_at(i), s_out[b])
        if not last and prefetch:
            pltpu.async_copy(ids_at(i + 2), idx[b], s_idx[b])

    def pair(g, carry):
        half(g, 0)
        half(g, 1)
        return carry

    pltpu.async_copy(ids_at(0), idx[0], s_idx[0])
    pltpu.async_copy(ids_at(1), idx[1], s_idx[1])
    pltpu.make_async_copy(ids_at(0), idx[0], s_idx[0]).wait()
    gat_start(0)

    half(0, 0, first=True)
    half(0, 1)
    lax.fori_loop(1, _NPAIR - 1, pair, 0)
    half(_NPAIR - 1, 0, prefetch=False)
    half(_NPAIR - 1, 1, last=True)
    pltpu.make_async_copy(rows[1], out_at(_NCHUNK - 1), s_out[1]).wait()


def kernel(intent_ids, table):
    ids = intent_ids.reshape(TOTAL).astype(jnp.int32)
    return _gather_kernel(ids, table)
